# baseline (speedup 1.0000x reference)
import functools

import jax
import jax.numpy as jnp
from jax import lax
from jax.experimental import pallas as pl
from jax.experimental.pallas import tpu as pltpu
from jax.experimental.pallas import tpu_sc as plsc

D = 16
BATCH = 16384
NW = 32
BPW = BATCH // NW
CHUNK = 128
NCH = BPW // CHUNK
N_NODE = 1000000
COFF = 0.1

_mesh = plsc.VectorSubcoreMesh(core_axis_name="c", subcore_axis_name="s")


@functools.partial(
    pl.kernel,
    out_type=jax.ShapeDtypeStruct((7 * BATCH,), jnp.float32),
    mesh=_mesh,
    compiler_params=pltpu.CompilerParams(
        use_tc_tiling_on_sc=False, needs_layout_passes=False
    ),
    scratch_types=[
        pltpu.VMEM((5, BPW), jnp.int32),
        pltpu.VMEM((D, 5 * BPW), jnp.int32),
        pltpu.VMEM((5, D, BPW), jnp.float32),
        pltpu.VMEM((7, BPW), jnp.float32),
        pltpu.SemaphoreType.DMA,
    ],
)
def _dots(pnp_hbm, table_hbm, out_hbm, pnp_v, idx_v, vals_v, acc_v, sem):
    wid = lax.axis_index("s") * 2 + lax.axis_index("c")
    base = wid * BPW
    for j in range(5):
        pltpu.sync_copy(pnp_hbm.at[pl.ds(j * BATCH + base, BPW)], pnp_v.at[j])

    def _build(c, carry):
        coff = c * N_NODE
        for j in range(5):
            for g in range(BPW // 16):
                v = pnp_v[j, pl.ds(g * 16, 16)] + coff
                idx_v[c, pl.ds(j * BPW + g * 16, 16)] = v
        return carry

    lax.fori_loop(0, D, _build, 0)

    def _fire(c, carry):
        copies = []
        for j in range(5):
            for k in range(NCH):
                copies.append(
                    pltpu.async_copy(
                        table_hbm.at[idx_v.at[c, pl.ds(j * BPW + k * CHUNK, CHUNK)]],
                        vals_v.at[j, c, pl.ds(k * CHUNK, CHUNK)],
                        sem,
                    )
                )
        for cp in copies:
            cp.wait()
        return carry

    lax.fori_loop(0, D, _fire, 0)

    def _mac(g, carry):
        sl = pl.ds(g * 16, 16)
        t = [None] * 7
        for c in range(D):
            vu = vals_v[0, c, sl]
            vpi = vals_v[1, c, sl]
            vni = vals_v[2, c, sl]
            vpe = vals_v[3, c, sl]
            vne = vals_v[4, c, sl]
            terms = (vu * vpi, vu * vni, vpi * vpi, vpe * vpe,
                     vne * vne, vpi * vpe, vpi * vne)
            if c == 0:
                t = list(terms)
            else:
                t = [a + b for a, b in zip(t, terms)]
        for q in range(7):
            acc_v[q, sl] = t[q]
        return carry

    lax.fori_loop(0, BPW // 16, _mac, 0)

    for q in range(7):
        pltpu.sync_copy(acc_v.at[q], out_hbm.at[pl.ds(q * BATCH + base, BPW)])


def _softplus(z):
    return jnp.maximum(z, 0.0) + jnp.log1p(jnp.exp(-jnp.abs(z)))


_ROWS = BATCH // 128


def _loss_body(s_ref, out_ref):
    def part(q):
        return s_ref[pl.ds(q * _ROWS, _ROWS), :]

    pos_pred = part(0)
    neg_pred = part(1)
    n_pi = part(2)
    n_pe = part(3)
    n_ne = part(4)
    a = part(5)
    b = part(6)

    cf = jnp.sum(_softplus(neg_pred - pos_pred))

    iv_pi = 1.0 / jnp.maximum(jnp.sqrt(n_pi), 1e-12)
    iv_pe = 1.0 / jnp.maximum(jnp.sqrt(n_pe), 1e-12)
    iv_ne = 1.0 / jnp.maximum(jnp.sqrt(n_ne), 1e-12)
    pos_reg = n_pi * iv_pi * iv_pi - 2.0 * a * iv_pi * iv_pe + n_pe * iv_pe * iv_pe
    neg_reg = n_pi * iv_pi * iv_pi - 2.0 * b * iv_pi * iv_ne + n_ne * iv_ne * iv_ne
    reg = jnp.sum(_softplus(neg_reg - pos_reg))

    out_ref[0, 0] = cf + COFF * reg


_loss = pl.pallas_call(
    _loss_body,
    out_shape=jax.ShapeDtypeStruct((1, 1), jnp.float32),
    in_specs=[pl.BlockSpec(memory_space=pltpu.VMEM)],
    out_specs=pl.BlockSpec(memory_space=pltpu.SMEM),
)


@jax.jit
def kernel(repr_x, pos_neg_pair_t):
    table_flat = repr_x.T.reshape(D * N_NODE)
    pnp_flat = pos_neg_pair_t.T.reshape(5 * BATCH)
    s = _dots(pnp_flat, table_flat)
    s2 = s.reshape(7 * _ROWS, 128)
    return _loss(s2)[0, 0]


# speedup vs baseline: 2.8222x; 2.8222x over previous
import functools

import jax
import jax.numpy as jnp
from jax import lax
from jax.experimental import pallas as pl
from jax.experimental.pallas import tpu as pltpu
from jax.experimental.pallas import tpu_sc as plsc

D = 16
BATCH = 16384
NW = 32
BPW = BATCH // NW
CHUNK = 128
NCH = BPW // CHUNK
COFF = 0.1
R = BATCH * D // 128

_mesh = plsc.VectorSubcoreMesh(core_axis_name="c", subcore_axis_name="s")

NR0 = 31232
CH0 = 2048
_PITCH = 2049


@functools.partial(
    pl.kernel,
    out_type=jax.ShapeDtypeStruct((125000, 128), jnp.float32),
    mesh=_mesh,
    compiler_params=pltpu.CompilerParams(
        use_tc_tiling_on_sc=True, needs_layout_passes=False
    ),
    scratch_types=[
        pltpu.VMEM((D, _PITCH), jnp.float32),
        pltpu.VMEM((CH0 // 8, 128), jnp.float32),
        pltpu.SemaphoreType.DMA,
    ],
)
def _detranspose(tin, tout, win_v, stage_v, sem):
    wid = lax.axis_index("s") * 2 + lax.axis_index("c")
    ibase = wid * NR0
    comp16 = lax.iota(jnp.int32, 16)
    zeros16 = comp16 * 0

    def emit_chunk(i0, ch):
        i0 = pl.multiple_of(i0, 128)
        copies = [
            pltpu.async_copy(
                tin.at[pl.ds(h, 8), pl.ds(i0, ch)],
                win_v.at[pl.ds(h, 8), pl.ds(0, ch)],
                sem,
            )
            for h in (0, 8)
        ]
        for cp in copies:
            cp.wait()

        def row(r, carry):
            for s in range(8):
                cols = zeros16 + (r * 8 + s)
                v = plsc.load_gather(win_v, [comp16, cols])
                stage_v[r, pl.ds(s * D, D)] = v
            return carry

        lax.fori_loop(0, ch // 8, row, 0)
        pltpu.sync_copy(
            stage_v.at[pl.ds(0, ch // 8), :],
            tout.at[pl.ds(pl.multiple_of(i0 // 8, 8), ch // 8), :],
        )

    for k in range(NR0 // CH0):
        emit_chunk(ibase + k * CH0, CH0)

    nfull = NR0 // CH0 * CH0

    @pl.when(wid < 31)
    def _tail_a():
        emit_chunk(ibase + nfull, NR0 - nfull)

    @pl.when(wid == 31)
    def _tail_b():
        emit_chunk(ibase + nfull, 1024)
        emit_chunk(ibase + nfull + 1024, 64)


@functools.partial(
    pl.kernel,
    out_type=jax.ShapeDtypeStruct((5 * R, 128), jnp.float32),
    mesh=_mesh,
    compiler_params=pltpu.CompilerParams(
        use_tc_tiling_on_sc=False, needs_layout_passes=False
    ),
    scratch_types=[
        pltpu.VMEM((5, BPW), jnp.int32),
        pltpu.VMEM((5, BPW, D), jnp.float32),
        pltpu.VMEM((5, BPW * D // 128, 128), jnp.float32),
        pltpu.SemaphoreType.DMA,
    ],
)
def _gather5(pnp_hbm, table_hbm, out_hbm, pnp_v, rows_v, stage_v, sem):
    wid = lax.axis_index("s") * 2 + lax.axis_index("c")
    base = wid * BPW
    for j in range(5):
        pltpu.sync_copy(pnp_hbm.at[pl.ds(j * BATCH + base, BPW)], pnp_v.at[j])
    copies = []
    for j in range(5):
        for k in range(NCH):
            copies.append(
                pltpu.async_copy(
                    table_hbm.at[pnp_v.at[j, pl.ds(k * CHUNK, CHUNK)]],
                    rows_v.at[j, pl.ds(k * CHUNK, CHUNK), :],
                    sem,
                )
            )
    for c in copies:
        c.wait()

    def _fmt(g, carry):
        for j in range(5):
            for t in range(8):
                stage_v[j, g, pl.ds(t * D, D)] = rows_v[j, g * 8 + t, :]
        return carry

    lax.fori_loop(0, BPW * D // 128, _fmt, 0)

    for j in range(5):
        pltpu.sync_copy(
            stage_v.at[j],
            out_hbm.at[pl.ds(j * R + wid * (BPW * D // 128), BPW * D // 128), :],
        )


def _softplus(z):
    return jnp.maximum(z, 0.0) + jnp.log1p(jnp.exp(-jnp.abs(z)))


def _loss_body(g_ref, out_ref):
    ru = g_ref[pl.ds(0 * R, R), :]
    rpi = g_ref[pl.ds(1 * R, R), :]
    rni = g_ref[pl.ds(2 * R, R), :]
    rpe = g_ref[pl.ds(3 * R, R), :]
    rne = g_ref[pl.ds(4 * R, R), :]

    d = lax.broadcasted_iota(jnp.int32, (128, 8), 0)
    k = lax.broadcasted_iota(jnp.int32, (128, 8), 1)
    sel = jnp.where(d // D == k, 1.0, 0.0).astype(jnp.float32)

    def gsum(x):
        return jnp.dot(x, sel, preferred_element_type=jnp.float32)

    pos_pred = gsum(ru * rpi)
    neg_pred = gsum(ru * rni)
    cf = jnp.sum(_softplus(neg_pred - pos_pred))

    n_pi = gsum(rpi * rpi)
    n_pe = gsum(rpe * rpe)
    n_ne = gsum(rne * rne)
    a = gsum(rpi * rpe)
    b = gsum(rpi * rne)
    iv_pi = 1.0 / jnp.maximum(jnp.sqrt(n_pi), 1e-12)
    iv_pe = 1.0 / jnp.maximum(jnp.sqrt(n_pe), 1e-12)
    iv_ne = 1.0 / jnp.maximum(jnp.sqrt(n_ne), 1e-12)
    pos_reg = n_pi * iv_pi * iv_pi - 2.0 * a * iv_pi * iv_pe + n_pe * iv_pe * iv_pe
    neg_reg = n_pi * iv_pi * iv_pi - 2.0 * b * iv_pi * iv_ne + n_ne * iv_ne * iv_ne
    reg = jnp.sum(_softplus(neg_reg - pos_reg))

    out_ref[0, 0] = cf + COFF * reg


_loss = pl.pallas_call(
    _loss_body,
    out_shape=jax.ShapeDtypeStruct((1, 1), jnp.float32),
    in_specs=[pl.BlockSpec(memory_space=pltpu.VMEM)],
    out_specs=pl.BlockSpec(memory_space=pltpu.SMEM),
)


@jax.jit
def kernel(repr_x, pos_neg_pair_t):
    rows128 = _detranspose(repr_x.T)
    table_rm = rows128.reshape(1000000, D)
    pnp_flat = pos_neg_pair_t.T.reshape(5 * BATCH)
    g = _gather5(pnp_flat, table_rm)
    return _loss(g)[0, 0]
